# baseline (speedup 1.0000x reference)
import functools

import jax
import jax.numpy as jnp
from jax import lax
from jax.experimental import pallas as pl
from jax.experimental.pallas import tpu as pltpu
from jax.experimental.pallas import tpu_sc as plsc

N_EMB = 256
D = 256
TILE = 1024
N_TOK = 32 * 1024

_info = plsc.get_sparse_core_info()
_NC, _NS = _info.num_cores, _info.num_subcores
NW = _NC * _NS
B_PER_W = N_TOK // NW
CHUNK = 128
NCHUNK = B_PER_W // CHUNK


def _dist_argmin_kernel(x_ref, emb_ref, idx_ref, perp_ref, hist_ref, en_ref):
    i = pl.program_id(0)
    n_steps = pl.num_programs(0)

    emb = emb_ref[...]

    @pl.when(i == 0)
    def _init():
        hist_ref[...] = jnp.zeros_like(hist_ref)
        en_ref[...] = jnp.sum(emb * emb, axis=0, keepdims=True)

    f = x_ref[...]
    sim = jnp.dot(f, emb, preferred_element_type=jnp.float32)
    row_norm = jnp.sum(f * f, axis=1, keepdims=True)
    distances = row_norm + en_ref[...] - 2.0 * sim
    idx = jnp.argmin(distances, axis=1)
    idx_ref[...] = idx[None, None, :]
    onehot = (lax.broadcasted_iota(jnp.int32, (TILE, N_EMB), 1)
              == idx[:, None]).astype(jnp.float32)
    hist_ref[...] += jnp.sum(onehot, axis=0, keepdims=True)

    @pl.when(i == n_steps - 1)
    def _finish():
        total = jnp.float32(n_steps * TILE)
        avg_probs = hist_ref[...] / total
        ent = jnp.sum(avg_probs * jnp.log(avg_probs + 1e-10))
        perp_ref[...] = jnp.exp(-ent)[None, None]


def _tc_indices(flat, embeddings):
    n = flat.shape[0]
    grid = (n // TILE,)
    idx, perp = pl.pallas_call(
        _dist_argmin_kernel,
        grid=grid,
        in_specs=[
            pl.BlockSpec((TILE, D), lambda i: (i, 0)),
            pl.BlockSpec((D, N_EMB), lambda i: (0, 0)),
        ],
        out_specs=[
            pl.BlockSpec((1, 1, TILE), lambda i: (i, 0, 0)),
            pl.BlockSpec((1, 1), lambda i: (0, 0)),
        ],
        out_shape=[
            jax.ShapeDtypeStruct((n // TILE, 1, TILE), jnp.int32),
            jax.ShapeDtypeStruct((1, 1), jnp.float32),
        ],
        scratch_shapes=[pltpu.VMEM((1, N_EMB), jnp.float32),
                        pltpu.VMEM((1, N_EMB), jnp.float32)],
    )(flat, embeddings)
    return idx, perp


def _sc_gather_body(emb_hbm, idx_hbm, out_hbm, idx_v, buf0, buf1, sem0, sem1):
    wid = lax.axis_index("s") * _NC + lax.axis_index("c")
    base = wid * B_PER_W
    pltpu.sync_copy(idx_hbm.at[wid], idx_v)
    bufs = (buf0, buf1)
    sems = (sem0, sem1)
    cps = [None, None]
    cps[0] = pltpu.async_copy(emb_hbm.at[idx_v.at[0]], bufs[0], sems[0])
    for c in range(1, NCHUNK):
        cps[c % 2] = pltpu.async_copy(emb_hbm.at[idx_v.at[c]],
                                      bufs[c % 2], sems[c % 2])
        cps[(c - 1) % 2].wait()
        pltpu.sync_copy(bufs[(c - 1) % 2],
                        out_hbm.at[pl.ds(base + (c - 1) * CHUNK, CHUNK)])
    last = NCHUNK - 1
    cps[last % 2].wait()
    pltpu.sync_copy(bufs[last % 2],
                    out_hbm.at[pl.ds(base + last * CHUNK, CHUNK)])


_sc_gather = functools.partial(
    pl.kernel,
    mesh=plsc.VectorSubcoreMesh(core_axis_name="c", subcore_axis_name="s"),
    out_type=jax.ShapeDtypeStruct((N_TOK, D), jnp.float32),
    scratch_types=[
        pltpu.VMEM((NCHUNK, CHUNK), jnp.int32),
        pltpu.VMEM((CHUNK, D), jnp.float32),
        pltpu.VMEM((CHUNK, D), jnp.float32),
        pltpu.SemaphoreType.DMA,
        pltpu.SemaphoreType.DMA,
    ],
)(_sc_gather_body)


@jax.jit
def kernel(x, embeddings):
    input_shape = x.shape
    flat = x.reshape(-1, D)
    idx3, perp = _tc_indices(flat, embeddings)
    idx = idx3.reshape(NW, NCHUNK, CHUNK)
    q = _sc_gather(embeddings, idx)
    return q.reshape(input_shape), perp[0, 0]


# speedup vs baseline: 1.0068x; 1.0068x over previous
import functools

import jax
import jax.numpy as jnp
from jax import lax
from jax.experimental import pallas as pl
from jax.experimental.pallas import tpu as pltpu
from jax.experimental.pallas import tpu_sc as plsc

N_EMB = 256
D = 256
TILE = 1024
N_TOK = 32 * 1024

_info = plsc.get_sparse_core_info()
_NC, _NS = _info.num_cores, _info.num_subcores
NW = _NC * _NS
B_PER_W = N_TOK // NW
CHUNK = 128
NCHUNK = B_PER_W // CHUNK


def _dist_argmin_kernel(x_ref, emb_ref, idx_ref, perp_ref, hist_ref, en_ref):
    i = pl.program_id(0)
    n_steps = pl.num_programs(0)

    emb = emb_ref[...]

    @pl.when(i == 0)
    def _init():
        hist_ref[...] = jnp.zeros_like(hist_ref)
        en_ref[...] = jnp.sum(emb * emb, axis=0, keepdims=True)

    f = x_ref[...]
    sim = jnp.dot(f, emb, preferred_element_type=jnp.float32)
    row_norm = jnp.sum(f * f, axis=1, keepdims=True)
    distances = row_norm + en_ref[...] - 2.0 * sim
    idx = jnp.argmin(distances, axis=1)
    idx_ref[...] = idx[None, None, :]
    onehot = (lax.broadcasted_iota(jnp.int32, (TILE, N_EMB), 1)
              == idx[:, None]).astype(jnp.float32)
    hist_ref[...] += jnp.sum(onehot, axis=0, keepdims=True)

    @pl.when(i == n_steps - 1)
    def _finish():
        total = jnp.float32(n_steps * TILE)
        avg_probs = hist_ref[...] / total
        ent = jnp.sum(avg_probs * jnp.log(avg_probs + 1e-10))
        perp_ref[...] = jnp.exp(-ent)[None, None]


def _tc_indices(flat, embeddings):
    n = flat.shape[0]
    grid = (n // TILE,)
    idx, perp = pl.pallas_call(
        _dist_argmin_kernel,
        grid=grid,
        in_specs=[
            pl.BlockSpec((TILE, D), lambda i: (i, 0)),
            pl.BlockSpec((D, N_EMB), lambda i: (0, 0)),
        ],
        out_specs=[
            pl.BlockSpec((1, 1, TILE), lambda i: (i, 0, 0)),
            pl.BlockSpec((1, 1), lambda i: (0, 0)),
        ],
        out_shape=[
            jax.ShapeDtypeStruct((n // TILE, 1, TILE), jnp.int32),
            jax.ShapeDtypeStruct((1, 1), jnp.float32),
        ],
        scratch_shapes=[pltpu.VMEM((1, N_EMB), jnp.float32),
                        pltpu.VMEM((1, N_EMB), jnp.float32)],
    )(flat, embeddings)
    return idx, perp


NBUF = 3


def _sc_gather_body(emb_hbm, idx_hbm, out_hbm, idx_v,
                    buf0, buf1, buf2, gs0, gs1, gs2, ss0, ss1, ss2):
    wid = lax.axis_index("s") * _NC + lax.axis_index("c")
    base = wid * B_PER_W
    pltpu.sync_copy(idx_hbm.at[wid], idx_v)
    bufs = (buf0, buf1, buf2)
    gsems = (gs0, gs1, gs2)
    ssems = (ss0, ss1, ss2)
    gcp = [None] * NBUF
    scp = [None] * NBUF
    for c in range(min(NBUF, NCHUNK)):
        gcp[c] = pltpu.async_copy(emb_hbm.at[idx_v.at[c]], bufs[c],
                                  gsems[c])
    for c in range(NCHUNK):
        b = c % NBUF
        gcp[b].wait()
        scp[b] = pltpu.async_copy(
            bufs[b], out_hbm.at[pl.ds(base + c * CHUNK, CHUNK)], ssems[b])
        nxt = c + NBUF
        if nxt < NCHUNK:
            scp[b].wait()
            gcp[b] = pltpu.async_copy(emb_hbm.at[idx_v.at[nxt]], bufs[b],
                                      gsems[b])
    for c in range(max(0, NCHUNK - NBUF), NCHUNK):
        scp[c % NBUF].wait()


_sc_gather = functools.partial(
    pl.kernel,
    mesh=plsc.VectorSubcoreMesh(core_axis_name="c", subcore_axis_name="s"),
    out_type=jax.ShapeDtypeStruct((N_TOK, D), jnp.float32),
    scratch_types=[
        pltpu.VMEM((NCHUNK, CHUNK), jnp.int32),
        pltpu.VMEM((CHUNK, D), jnp.float32),
        pltpu.VMEM((CHUNK, D), jnp.float32),
        pltpu.VMEM((CHUNK, D), jnp.float32),
        pltpu.SemaphoreType.DMA,
        pltpu.SemaphoreType.DMA,
        pltpu.SemaphoreType.DMA,
        pltpu.SemaphoreType.DMA,
        pltpu.SemaphoreType.DMA,
        pltpu.SemaphoreType.DMA,
    ],
)(_sc_gather_body)


@jax.jit
def kernel(x, embeddings):
    input_shape = x.shape
    flat = x.reshape(-1, D)
    idx3, perp = _tc_indices(flat, embeddings)
    idx = idx3.reshape(NW, NCHUNK, CHUNK)
    q = _sc_gather(embeddings, idx)
    return q.reshape(input_shape), perp[0, 0]


# speedup vs baseline: 1.0497x; 1.0427x over previous
import functools

import jax
import jax.numpy as jnp
from jax import lax
from jax.experimental import pallas as pl
from jax.experimental.pallas import tpu as pltpu
from jax.experimental.pallas import tpu_sc as plsc

N_EMB = 256
D = 256
TILE = 1024
N_TOK = 32 * 1024

_info = plsc.get_sparse_core_info()
_NC, _NS = _info.num_cores, _info.num_subcores
NW = _NC * _NS
B_PER_W = N_TOK // NW
CHUNK = 128
NCHUNK = B_PER_W // CHUNK


def _dist_argmin_kernel(x_ref, emb_ref, idx_ref, perp_ref, hist_ref, en_ref):
    i = pl.program_id(0)
    n_steps = pl.num_programs(0)

    emb = emb_ref[...]

    @pl.when(i == 0)
    def _init():
        hist_ref[...] = jnp.zeros_like(hist_ref)
        en_ref[...] = jnp.sum(emb * emb, axis=0, keepdims=True)

    f = x_ref[...]
    sim = jnp.dot(f, emb, preferred_element_type=jnp.float32)
    row_norm = jnp.sum(f * f, axis=1, keepdims=True)
    distances = row_norm + en_ref[...] - 2.0 * sim
    lane_f = lax.broadcasted_iota(jnp.int32, (TILE, N_EMB), 1).astype(
        jnp.float32)
    dmin = jnp.min(distances, axis=1, keepdims=True)
    idx_f = jnp.min(jnp.where(distances <= dmin, lane_f, jnp.float32(N_EMB)),
                    axis=1)
    idx = idx_f.astype(jnp.int32)
    idx_ref[...] = idx[None, None, :]
    onehot = (lane_f == idx_f[:, None]).astype(jnp.float32)
    hist_ref[...] += jnp.sum(onehot, axis=0, keepdims=True)

    @pl.when(i == n_steps - 1)
    def _finish():
        total = jnp.float32(n_steps * TILE)
        avg_probs = hist_ref[...] / total
        ent = jnp.sum(avg_probs * jnp.log(avg_probs + 1e-10))
        perp_ref[...] = jnp.exp(-ent)[None, None]


def _tc_indices(flat, embeddings):
    n = flat.shape[0]
    grid = (n // TILE,)
    idx, perp = pl.pallas_call(
        _dist_argmin_kernel,
        grid=grid,
        in_specs=[
            pl.BlockSpec((TILE, D), lambda i: (i, 0)),
            pl.BlockSpec((D, N_EMB), lambda i: (0, 0)),
        ],
        out_specs=[
            pl.BlockSpec((1, 1, TILE), lambda i: (i, 0, 0)),
            pl.BlockSpec((1, 1), lambda i: (0, 0)),
        ],
        out_shape=[
            jax.ShapeDtypeStruct((n // TILE, 1, TILE), jnp.int32),
            jax.ShapeDtypeStruct((1, 1), jnp.float32),
        ],
        scratch_shapes=[pltpu.VMEM((1, N_EMB), jnp.float32),
                        pltpu.VMEM((1, N_EMB), jnp.float32)],
    )(flat, embeddings)
    return idx, perp


NBUF = 3


def _sc_gather_body(emb_hbm, idx_hbm, out_hbm, idx_v,
                    buf0, buf1, buf2, gs0, gs1, gs2, ss0, ss1, ss2):
    wid = lax.axis_index("s") * _NC + lax.axis_index("c")
    base = wid * B_PER_W
    pltpu.sync_copy(idx_hbm.at[wid], idx_v)
    bufs = (buf0, buf1, buf2)
    gsems = (gs0, gs1, gs2)
    ssems = (ss0, ss1, ss2)
    gcp = [None] * NBUF
    scp = [None] * NBUF
    for c in range(min(NBUF, NCHUNK)):
        gcp[c] = pltpu.async_copy(emb_hbm.at[idx_v.at[c]], bufs[c],
                                  gsems[c])
    for c in range(NCHUNK):
        b = c % NBUF
        gcp[b].wait()
        scp[b] = pltpu.async_copy(
            bufs[b], out_hbm.at[pl.ds(base + c * CHUNK, CHUNK)], ssems[b])
        nxt = c + NBUF
        if nxt < NCHUNK:
            scp[b].wait()
            gcp[b] = pltpu.async_copy(emb_hbm.at[idx_v.at[nxt]], bufs[b],
                                      gsems[b])
    for c in range(max(0, NCHUNK - NBUF), NCHUNK):
        scp[c % NBUF].wait()


_sc_gather = functools.partial(
    pl.kernel,
    mesh=plsc.VectorSubcoreMesh(core_axis_name="c", subcore_axis_name="s"),
    out_type=jax.ShapeDtypeStruct((N_TOK, D), jnp.float32),
    scratch_types=[
        pltpu.VMEM((NCHUNK, CHUNK), jnp.int32),
        pltpu.VMEM((CHUNK, D), jnp.float32),
        pltpu.VMEM((CHUNK, D), jnp.float32),
        pltpu.VMEM((CHUNK, D), jnp.float32),
        pltpu.SemaphoreType.DMA,
        pltpu.SemaphoreType.DMA,
        pltpu.SemaphoreType.DMA,
        pltpu.SemaphoreType.DMA,
        pltpu.SemaphoreType.DMA,
        pltpu.SemaphoreType.DMA,
    ],
)(_sc_gather_body)


@jax.jit
def kernel(x, embeddings):
    input_shape = x.shape
    flat = x.reshape(-1, D)
    idx3, perp = _tc_indices(flat, embeddings)
    idx = idx3.reshape(NW, NCHUNK, CHUNK)
    q = _sc_gather(embeddings, idx)
    return q.reshape(input_shape), perp[0, 0]


# speedup vs baseline: 1.1036x; 1.0513x over previous
import functools

import jax
import jax.numpy as jnp
from jax import lax
from jax.experimental import pallas as pl
from jax.experimental.pallas import tpu as pltpu
from jax.experimental.pallas import tpu_sc as plsc

N_EMB = 256
D = 256
TILE = 1024
N_TOK = 32 * 1024
HALF = N_TOK // 2

_info = plsc.get_sparse_core_info()
_NC, _NS = _info.num_cores, _info.num_subcores
NW = _NC * _NS
B_PER_W = HALF // NW
CHUNK = 128
NCHUNK = B_PER_W // CHUNK


def _distance_argmin(f, emb, en):
    sim = jnp.dot(f, emb, preferred_element_type=jnp.float32)
    row_norm = jnp.sum(f * f, axis=1, keepdims=True)
    distances = row_norm + en - 2.0 * sim
    lane_f = lax.broadcasted_iota(jnp.int32, (TILE, N_EMB), 1).astype(
        jnp.float32)
    dmin = jnp.min(distances, axis=1, keepdims=True)
    idx_f = jnp.min(jnp.where(distances <= dmin, lane_f, jnp.float32(N_EMB)),
                    axis=1, keepdims=True)
    onehot = (lane_f == idx_f).astype(jnp.float32)
    return idx_f, onehot


def _idx_kernel(x_ref, emb_ref, idx_ref, hist_out_ref, hist_ref, en_ref):
    i = pl.program_id(0)
    n_steps = pl.num_programs(0)
    emb = emb_ref[...]

    @pl.when(i == 0)
    def _init():
        hist_ref[...] = jnp.zeros_like(hist_ref)
        en_ref[...] = jnp.sum(emb * emb, axis=0, keepdims=True)

    idx_f, onehot = _distance_argmin(x_ref[...], emb, en_ref[...])
    idx_ref[...] = idx_f.astype(jnp.int32)
    hist_ref[...] += jnp.sum(onehot, axis=0, keepdims=True)

    @pl.when(i == n_steps - 1)
    def _finish():
        hist_out_ref[...] = hist_ref[...]


def _quant_kernel(x_ref, emb_ref, hist1_ref, q_ref, perp_ref, hist_ref,
                  en_ref):
    i = pl.program_id(0)
    n_steps = pl.num_programs(0)
    emb = emb_ref[...]

    @pl.when(i == 0)
    def _init():
        hist_ref[...] = hist1_ref[...]
        en_ref[...] = jnp.sum(emb * emb, axis=0, keepdims=True)

    _, onehot = _distance_argmin(x_ref[...], emb, en_ref[...])
    q_ref[...] = jnp.dot(onehot, emb, preferred_element_type=jnp.float32)
    hist_ref[...] += jnp.sum(onehot, axis=0, keepdims=True)

    @pl.when(i == n_steps - 1)
    def _finish():
        avg_probs = hist_ref[...] / jnp.float32(N_TOK)
        ent = jnp.sum(avg_probs * jnp.log(avg_probs + 1e-10))
        perp_ref[...] = jnp.exp(-ent)[None, None]


def _tc_phase1(flat, embeddings):
    idx, hist = pl.pallas_call(
        _idx_kernel,
        grid=(HALF // TILE,),
        in_specs=[
            pl.BlockSpec((TILE, D), lambda i: (i, 0)),
            pl.BlockSpec((D, N_EMB), lambda i: (0, 0)),
        ],
        out_specs=[
            pl.BlockSpec((TILE, 1), lambda i: (i, 0)),
            pl.BlockSpec((1, N_EMB), lambda i: (0, 0)),
        ],
        out_shape=[
            jax.ShapeDtypeStruct((HALF, 1), jnp.int32),
            jax.ShapeDtypeStruct((1, N_EMB), jnp.float32),
        ],
        scratch_shapes=[pltpu.VMEM((1, N_EMB), jnp.float32),
                        pltpu.VMEM((1, N_EMB), jnp.float32)],
    )(flat, embeddings)
    return idx, hist


def _tc_phase2(flat, embeddings, hist1):
    nhalf = HALF // TILE
    q2, perp = pl.pallas_call(
        _quant_kernel,
        grid=(nhalf,),
        in_specs=[
            pl.BlockSpec((TILE, D), lambda i: (i + nhalf, 0)),
            pl.BlockSpec((D, N_EMB), lambda i: (0, 0)),
            pl.BlockSpec((1, N_EMB), lambda i: (0, 0)),
        ],
        out_specs=[
            pl.BlockSpec((TILE, D), lambda i: (i, 0)),
            pl.BlockSpec((1, 1), lambda i: (0, 0)),
        ],
        out_shape=[
            jax.ShapeDtypeStruct((HALF, D), jnp.float32),
            jax.ShapeDtypeStruct((1, 1), jnp.float32),
        ],
        scratch_shapes=[pltpu.VMEM((1, N_EMB), jnp.float32),
                        pltpu.VMEM((1, N_EMB), jnp.float32)],
    )(flat, embeddings, hist1)
    return q2, perp


NBUF = 3


def _sc_gather_body(emb_hbm, idx_hbm, out_hbm, idx_v,
                    buf0, buf1, buf2, gs0, gs1, gs2, ss0, ss1, ss2):
    wid = lax.axis_index("s") * _NC + lax.axis_index("c")
    base = wid * B_PER_W
    pltpu.sync_copy(idx_hbm.at[wid], idx_v)
    bufs = (buf0, buf1, buf2)
    gsems = (gs0, gs1, gs2)
    ssems = (ss0, ss1, ss2)
    gcp = [None] * NBUF
    scp = [None] * NBUF
    for c in range(min(NBUF, NCHUNK)):
        gcp[c] = pltpu.async_copy(emb_hbm.at[idx_v.at[c]], bufs[c],
                                  gsems[c])
    for c in range(NCHUNK):
        b = c % NBUF
        gcp[b].wait()
        scp[b] = pltpu.async_copy(
            bufs[b], out_hbm.at[pl.ds(base + c * CHUNK, CHUNK)], ssems[b])
        nxt = c + NBUF
        if nxt < NCHUNK:
            scp[b].wait()
            gcp[b] = pltpu.async_copy(emb_hbm.at[idx_v.at[nxt]], bufs[b],
                                      gsems[b])
    for c in range(max(0, NCHUNK - NBUF), NCHUNK):
        scp[c % NBUF].wait()


_sc_gather = functools.partial(
    pl.kernel,
    mesh=plsc.VectorSubcoreMesh(core_axis_name="c", subcore_axis_name="s"),
    out_type=jax.ShapeDtypeStruct((HALF, D), jnp.float32),
    scratch_types=[
        pltpu.VMEM((NCHUNK, CHUNK), jnp.int32),
        pltpu.VMEM((CHUNK, D), jnp.float32),
        pltpu.VMEM((CHUNK, D), jnp.float32),
        pltpu.VMEM((CHUNK, D), jnp.float32),
        pltpu.SemaphoreType.DMA,
        pltpu.SemaphoreType.DMA,
        pltpu.SemaphoreType.DMA,
        pltpu.SemaphoreType.DMA,
        pltpu.SemaphoreType.DMA,
        pltpu.SemaphoreType.DMA,
    ],
)(_sc_gather_body)


@jax.jit
def kernel(x, embeddings):
    input_shape = x.shape
    flat = x.reshape(-1, D)
    idx1, hist1 = _tc_phase1(flat, embeddings)
    q1 = _sc_gather(embeddings, idx1.reshape(NW, NCHUNK, CHUNK))
    q2, perp = _tc_phase2(flat, embeddings, hist1)
    q = jnp.concatenate([q1, q2], axis=0)
    return q.reshape(input_shape), perp[0, 0]


# speedup vs baseline: 2.4782x; 2.2455x over previous
import jax
import jax.numpy as jnp
from jax import lax
from jax.experimental import pallas as pl
from jax.experimental.pallas import tpu as pltpu

N_EMB = 256
D = 256
TILE = 1024


def _vq_kernel(x_ref, emb_ref, q_ref, perp_ref, hist_ref, en_ref):
    i = pl.program_id(0)
    n_steps = pl.num_programs(0)

    emb = emb_ref[...]

    @pl.when(i == 0)
    def _init():
        hist_ref[...] = jnp.zeros_like(hist_ref)
        en_ref[...] = jnp.sum(emb * emb, axis=0, keepdims=True)

    f = x_ref[...]
    sim = jnp.dot(f, emb, preferred_element_type=jnp.float32)
    row_norm = jnp.sum(f * f, axis=1, keepdims=True)
    distances = row_norm + en_ref[...] - 2.0 * sim
    lane_f = lax.broadcasted_iota(jnp.int32, (TILE, N_EMB), 1).astype(
        jnp.float32)
    dmin = jnp.min(distances, axis=1, keepdims=True)
    idx_f = jnp.min(jnp.where(distances <= dmin, lane_f, jnp.float32(N_EMB)),
                    axis=1, keepdims=True)
    onehot = (lane_f == idx_f).astype(jnp.float32)
    q_ref[...] = jnp.dot(onehot, emb, preferred_element_type=jnp.float32)
    hist_ref[...] += jnp.sum(onehot, axis=0, keepdims=True)

    @pl.when(i == n_steps - 1)
    def _finish():
        total = jnp.float32(n_steps * TILE)
        avg_probs = hist_ref[...] / total
        ent = jnp.sum(avg_probs * jnp.log(avg_probs + 1e-10))
        perp_ref[...] = jnp.exp(-ent)[None, None]


@jax.jit
def kernel(x, embeddings):
    input_shape = x.shape
    flat = x.reshape(-1, D)
    n = flat.shape[0]
    grid = (n // TILE,)
    q, perp = pl.pallas_call(
        _vq_kernel,
        grid=grid,
        in_specs=[
            pl.BlockSpec((TILE, D), lambda i: (i, 0)),
            pl.BlockSpec((D, N_EMB), lambda i: (0, 0)),
        ],
        out_specs=[
            pl.BlockSpec((TILE, D), lambda i: (i, 0)),
            pl.BlockSpec((1, 1), lambda i: (0, 0)),
        ],
        out_shape=[
            jax.ShapeDtypeStruct((n, D), jnp.float32),
            jax.ShapeDtypeStruct((1, 1), jnp.float32),
        ],
        scratch_shapes=[pltpu.VMEM((1, N_EMB), jnp.float32),
                        pltpu.VMEM((1, N_EMB), jnp.float32)],
    )(flat, embeddings)
    return q.reshape(input_shape), perp[0, 0]


# speedup vs baseline: 3.2526x; 1.3125x over previous
import jax
import jax.numpy as jnp
from jax import lax
from jax.experimental import pallas as pl
from jax.experimental.pallas import tpu as pltpu

N_EMB = 256
D = 256
TILE = 2048


def _vq_kernel(x_ref, emb_ref, q_ref, perp_ref, hist_ref, en_ref):
    i = pl.program_id(0)
    n_steps = pl.num_programs(0)

    emb = emb_ref[...]

    @pl.when(i == 0)
    def _init():
        hist_ref[...] = jnp.zeros_like(hist_ref)
        en_ref[...] = jnp.sum(emb * emb, axis=0, keepdims=True)

    f = x_ref[...]
    sim = jnp.dot(f, emb, preferred_element_type=jnp.float32)
    row_norm = jnp.sum(f * f, axis=1, keepdims=True)
    distances = row_norm + en_ref[...] - 2.0 * sim
    lane_f = lax.broadcasted_iota(jnp.int32, (TILE, N_EMB), 1).astype(
        jnp.float32)
    dmin = jnp.min(distances, axis=1, keepdims=True)
    idx_f = jnp.min(jnp.where(distances <= dmin, lane_f, jnp.float32(N_EMB)),
                    axis=1, keepdims=True)
    onehot = (lane_f == idx_f).astype(jnp.float32)
    q_ref[...] = jnp.dot(onehot, emb, preferred_element_type=jnp.float32)
    hist_ref[...] += jnp.sum(onehot, axis=0, keepdims=True)

    @pl.when(i == n_steps - 1)
    def _finish():
        total = jnp.float32(n_steps * TILE)
        avg_probs = hist_ref[...] / total
        ent = jnp.sum(avg_probs * jnp.log(avg_probs + 1e-10))
        perp_ref[...] = jnp.exp(-ent)[None, None]


@jax.jit
def kernel(x, embeddings):
    input_shape = x.shape
    flat = x.reshape(-1, D)
    n = flat.shape[0]
    grid = (n // TILE,)
    q, perp = pl.pallas_call(
        _vq_kernel,
        grid=grid,
        in_specs=[
            pl.BlockSpec((TILE, D), lambda i: (i, 0)),
            pl.BlockSpec((D, N_EMB), lambda i: (0, 0)),
        ],
        out_specs=[
            pl.BlockSpec((TILE, D), lambda i: (i, 0)),
            pl.BlockSpec((1, 1), lambda i: (0, 0)),
        ],
        out_shape=[
            jax.ShapeDtypeStruct((n, D), jnp.float32),
            jax.ShapeDtypeStruct((1, 1), jnp.float32),
        ],
        scratch_shapes=[pltpu.VMEM((1, N_EMB), jnp.float32),
                        pltpu.VMEM((1, N_EMB), jnp.float32)],
    )(flat, embeddings)
    return q.reshape(input_shape), perp[0, 0]


# speedup vs baseline: 3.7973x; 1.1675x over previous
import jax
import jax.numpy as jnp
from jax import lax
from jax.experimental import pallas as pl
from jax.experimental.pallas import tpu as pltpu

N_EMB = 256
D = 256
TILE = 4096


def _vq_kernel(x_ref, emb_ref, q_ref, perp_ref, hist_ref, en_ref):
    i = pl.program_id(0)
    n_steps = pl.num_programs(0)

    emb = emb_ref[...]

    @pl.when(i == 0)
    def _init():
        hist_ref[...] = jnp.zeros_like(hist_ref)
        en_ref[...] = jnp.sum(emb * emb, axis=0, keepdims=True)

    f = x_ref[...]
    sim = jnp.dot(f, emb, preferred_element_type=jnp.float32)
    row_norm = jnp.sum(f * f, axis=1, keepdims=True)
    distances = row_norm + en_ref[...] - 2.0 * sim
    lane_f = lax.broadcasted_iota(jnp.int32, (TILE, N_EMB), 1).astype(
        jnp.float32)
    dmin = jnp.min(distances, axis=1, keepdims=True)
    idx_f = jnp.min(jnp.where(distances <= dmin, lane_f, jnp.float32(N_EMB)),
                    axis=1, keepdims=True)
    onehot = (lane_f == idx_f).astype(jnp.float32)
    q_ref[...] = jnp.dot(onehot, emb, preferred_element_type=jnp.float32)
    hist_ref[...] += jnp.sum(onehot, axis=0, keepdims=True)

    @pl.when(i == n_steps - 1)
    def _finish():
        total = jnp.float32(n_steps * TILE)
        avg_probs = hist_ref[...] / total
        ent = jnp.sum(avg_probs * jnp.log(avg_probs + 1e-10))
        perp_ref[...] = jnp.exp(-ent)[None, None]


@jax.jit
def kernel(x, embeddings):
    input_shape = x.shape
    flat = x.reshape(-1, D)
    n = flat.shape[0]
    grid = (n // TILE,)
    q, perp = pl.pallas_call(
        _vq_kernel,
        grid=grid,
        in_specs=[
            pl.BlockSpec((TILE, D), lambda i: (i, 0)),
            pl.BlockSpec((D, N_EMB), lambda i: (0, 0)),
        ],
        out_specs=[
            pl.BlockSpec((TILE, D), lambda i: (i, 0)),
            pl.BlockSpec((1, 1), lambda i: (0, 0)),
        ],
        out_shape=[
            jax.ShapeDtypeStruct((n, D), jnp.float32),
            jax.ShapeDtypeStruct((1, 1), jnp.float32),
        ],
        scratch_shapes=[pltpu.VMEM((1, N_EMB), jnp.float32),
                        pltpu.VMEM((1, N_EMB), jnp.float32)],
    )(flat, embeddings)
    return q.reshape(input_shape), perp[0, 0]


# speedup vs baseline: 3.9365x; 1.0367x over previous
import jax
import jax.numpy as jnp
from jax import lax
from jax.experimental import pallas as pl
from jax.experimental.pallas import tpu as pltpu

N_EMB = 256
D = 256
TILE = 8192


def _vq_kernel(x_ref, emb_ref, q_ref, perp_ref, hist_ref, en_ref):
    i = pl.program_id(0)
    n_steps = pl.num_programs(0)

    emb = emb_ref[...]

    @pl.when(i == 0)
    def _init():
        hist_ref[...] = jnp.zeros_like(hist_ref)
        en_ref[...] = jnp.sum(emb * emb, axis=0, keepdims=True)

    f = x_ref[...]
    sim = jnp.dot(f, emb, preferred_element_type=jnp.float32)
    row_norm = jnp.sum(f * f, axis=1, keepdims=True)
    distances = row_norm + en_ref[...] - 2.0 * sim
    lane_f = lax.broadcasted_iota(jnp.int32, (TILE, N_EMB), 1).astype(
        jnp.float32)
    dmin = jnp.min(distances, axis=1, keepdims=True)
    idx_f = jnp.min(jnp.where(distances <= dmin, lane_f, jnp.float32(N_EMB)),
                    axis=1, keepdims=True)
    onehot = (lane_f == idx_f).astype(jnp.float32)
    q_ref[...] = jnp.dot(onehot, emb, preferred_element_type=jnp.float32)
    hist_ref[...] += jnp.sum(onehot, axis=0, keepdims=True)

    @pl.when(i == n_steps - 1)
    def _finish():
        total = jnp.float32(n_steps * TILE)
        avg_probs = hist_ref[...] / total
        ent = jnp.sum(avg_probs * jnp.log(avg_probs + 1e-10))
        perp_ref[...] = jnp.exp(-ent)[None, None]


@jax.jit
def kernel(x, embeddings):
    input_shape = x.shape
    flat = x.reshape(-1, D)
    n = flat.shape[0]
    grid = (n // TILE,)
    q, perp = pl.pallas_call(
        _vq_kernel,
        grid=grid,
        in_specs=[
            pl.BlockSpec((TILE, D), lambda i: (i, 0)),
            pl.BlockSpec((D, N_EMB), lambda i: (0, 0)),
        ],
        out_specs=[
            pl.BlockSpec((TILE, D), lambda i: (i, 0)),
            pl.BlockSpec((1, 1), lambda i: (0, 0)),
        ],
        out_shape=[
            jax.ShapeDtypeStruct((n, D), jnp.float32),
            jax.ShapeDtypeStruct((1, 1), jnp.float32),
        ],
        scratch_shapes=[pltpu.VMEM((1, N_EMB), jnp.float32),
                        pltpu.VMEM((1, N_EMB), jnp.float32)],
    )(flat, embeddings)
    return q.reshape(input_shape), perp[0, 0]
